# baseline (speedup 1.0000x reference)
import functools

import jax
import jax.numpy as jnp
from jax.experimental import pallas as pl
from jax.experimental.pallas import tpu as pltpu

_NB = 4
_K = 1024
_D = 128
_ROWS = 8 * 576
_TILE = 4608


_KC = 256


def _vq_body(z_ref, cb_ref, c2_ref, codes_ref, inds_ref, comm_ref):
    j = pl.program_id(1)
    z = z_ref[...]
    cb = cb_ref[0]
    c2 = c2_ref[0, 0, :]
    z2 = jnp.sum(z * z, axis=1, keepdims=True)
    lane_c = jax.lax.broadcasted_iota(jnp.int32, (1, _KC), 1)
    lane_cf = lane_c.astype(jnp.float32)
    ms, idxs = [], []
    for kc in range(_K // _KC):
        cbc = cb[kc * _KC:(kc + 1) * _KC, :]
        dots = jnp.dot(z, cbc.T, preferred_element_type=jnp.float32)
        dist = z2 - 2.0 * dots + c2[None, kc * _KC:(kc + 1) * _KC]
        ms.append(jnp.min(dist, axis=1))
        idxs.append(jnp.min(jnp.where(dist == ms[-1][:, None], lane_cf,
                                      jnp.float32(_KC)), axis=1))
    m = ms[0]
    for kc in range(1, _K // _KC):
        m = jnp.minimum(m, ms[kc])
    idx_f = jnp.zeros_like(m)
    for kc in reversed(range(_K // _KC)):
        idx_f = jnp.where(ms[kc] == m, jnp.float32(kc * _KC) + idxs[kc],
                          idx_f)
    idx = idx_f.astype(jnp.int32)
    lane_row = jax.lax.broadcasted_iota(jnp.int32, (1, _K), 1)
    onehot = (lane_row == idx[:, None]).astype(jnp.bfloat16)
    q = jnp.dot(onehot, cb.astype(jnp.bfloat16),
                preferred_element_type=jnp.float32)
    codes_ref[...] = q
    inds_ref[0, 0, :] = idx
    s = jnp.sum(m.reshape(_TILE // _D, _D), axis=0)

    @pl.when(j == 0)
    def _init():
        comm_ref[0, 0, :] = s

    @pl.when(j > 0)
    def _acc():
        comm_ref[0, 0, :] += s


@functools.partial(jax.jit)
def kernel(x, codebooks):
    b, n, D = x.shape
    xr = x.reshape(b * n, D)
    c2in = jnp.sum(codebooks * codebooks, axis=-1)[:, None, :]
    ntiles = _ROWS // _TILE
    codes, inds3, comm = pl.pallas_call(
        _vq_body,
        grid=(_NB, ntiles),
        in_specs=[
            pl.BlockSpec((_TILE, _D), lambda i, j: (j, i)),
            pl.BlockSpec((1, _K, _D), lambda i, j: (i, 0, 0)),
            pl.BlockSpec((1, 1, _K), lambda i, j: (i, 0, 0)),
        ],
        out_specs=[
            pl.BlockSpec((_TILE, _D), lambda i, j: (j, i)),
            pl.BlockSpec((1, 1, _TILE), lambda i, j: (i, 0, j)),
            pl.BlockSpec((1, 1, _D), lambda i, j: (i, 0, 0)),
        ],
        out_shape=[
            jax.ShapeDtypeStruct((_ROWS, _NB * _D), jnp.float32),
            jax.ShapeDtypeStruct((_NB, 1, _ROWS), jnp.int32),
            jax.ShapeDtypeStruct((_NB, 1, _D), jnp.float32),
        ],
    )(xr, codebooks, c2in)
    codes = codes.reshape(b, n, D)
    inds = inds3.reshape(_NB, b, n).transpose(1, 2, 0)
    commits = jnp.sum(comm[:, 0, :], axis=-1) / jnp.float32(_ROWS * _D)
    return (codes, inds, commits)


# speedup vs baseline: 1.1154x; 1.1154x over previous
import functools

import jax
import jax.numpy as jnp
from jax.experimental import pallas as pl

_NB = 4
_K = 1024
_D = 128
_ROWS = 8 * 576


def _vq_body(z_ref, cb_ref, codes_ref, inds_ref, comm_ref):
    i = pl.program_id(0)
    z = z_ref[...]
    cb = cb_ref[0]
    c2 = jnp.sum(cb * cb, axis=1)
    dots = jnp.dot(z, cb.T, preferred_element_type=jnp.float32)
    z2 = jnp.sum(z * z, axis=1, keepdims=True)
    dist = z2 - 2.0 * dots + c2[None, :]
    m = jnp.min(dist, axis=1)
    lane_row = jax.lax.broadcasted_iota(jnp.int32, (1, _K), 1)
    lane_f = lane_row.astype(jnp.float32)
    idx_f = jnp.min(jnp.where(dist == m[:, None], lane_f,
                              jnp.float32(_K)), axis=1)
    idx = idx_f.astype(jnp.int32)
    onehot = (lane_row == idx[:, None]).astype(jnp.bfloat16)
    q = jnp.dot(onehot, cb.astype(jnp.bfloat16),
                preferred_element_type=jnp.float32)
    codes_ref[...] = q

    col = jax.lax.broadcasted_iota(jnp.int32, (1, _NB), 1)
    @pl.when(i == 0)
    def _init_inds():
        inds_ref[...] = jnp.zeros((_ROWS, _NB), jnp.int32)
    inds_ref[...] = jnp.where(col == i, idx[:, None], inds_ref[...])

    s = jnp.sum(m) / jnp.float32(_ROWS * _D)
    lane4 = jax.lax.broadcasted_iota(jnp.int32, (1, _NB), 1)
    @pl.when(i == 0)
    def _init_comm():
        comm_ref[...] = jnp.zeros((1, _NB), jnp.float32)
    comm_ref[...] = jnp.where(lane4 == i, s, comm_ref[...])


@functools.partial(jax.jit)
def kernel(x, codebooks):
    b, n, D = x.shape
    xr = x.reshape(b * n, D)
    codes, inds, comm = pl.pallas_call(
        _vq_body,
        grid=(_NB,),
        in_specs=[
            pl.BlockSpec((_ROWS, _D), lambda i: (0, i)),
            pl.BlockSpec((1, _K, _D), lambda i: (i, 0, 0)),
        ],
        out_specs=[
            pl.BlockSpec((_ROWS, _D), lambda i: (0, i)),
            pl.BlockSpec((_ROWS, _NB), lambda i: (0, 0)),
            pl.BlockSpec((1, _NB), lambda i: (0, 0)),
        ],
        out_shape=[
            jax.ShapeDtypeStruct((_ROWS, _NB * _D), jnp.float32),
            jax.ShapeDtypeStruct((_ROWS, _NB), jnp.int32),
            jax.ShapeDtypeStruct((1, _NB), jnp.float32),
        ],
    )(xr, codebooks)
    return (codes.reshape(b, n, D), inds.reshape(b, n, _NB),
            comm.reshape(_NB))
